# baseline (speedup 1.0000x reference)
import functools

import jax
import jax.numpy as jnp
from jax import lax
from jax.experimental import pallas as pl
from jax.experimental.pallas import tpu as pltpu
from jax.experimental.pallas import tpu_sc as plsc

NENTITY = 1000000
NRELATION = 1000000
HIDDEN_DIM = 64
GAMMA = 12.0
BATCH = 16384

_INFO = plsc.get_sparse_core_info()
_NC = _INFO.num_cores
_NS = _INFO.num_subcores
_NW = _NC * _NS
_BPW = BATCH // _NW
_CHUNK = 128
_NCHUNK = _BPW // _CHUNK
_GROUPS = _BPW // 16


def _make_kernel():
    mesh = plsc.VectorSubcoreMesh(core_axis_name="c", subcore_axis_name="s")

    @functools.partial(
        pl.kernel,
        mesh=mesh,
        out_type=jax.ShapeDtypeStruct((BATCH,), jnp.float32),
        scratch_types=[
            pltpu.VMEM((_NCHUNK, _CHUNK), jnp.int32),
            pltpu.VMEM((_NCHUNK, _CHUNK), jnp.int32),
            pltpu.VMEM((_NCHUNK, _CHUNK), jnp.int32),
            pltpu.VMEM((_BPW, HIDDEN_DIM), jnp.float32),
            pltpu.VMEM((_BPW, HIDDEN_DIM), jnp.float32),
            pltpu.VMEM((_BPW, HIDDEN_DIM), jnp.float32),
            pltpu.VMEM((_BPW,), jnp.float32),
            pltpu.SemaphoreType.DMA,
        ],
        compiler_params=pltpu.CompilerParams(
            needs_layout_passes=False, use_tc_tiling_on_sc=False),
    )
    def kge_score(h_hbm, r_hbm, t_hbm, ent_hbm, rel_hbm, out_hbm,
                  idx_h, idx_r, idx_t, rows_h, rows_r, rows_t, out_v, sem):
        wid = lax.axis_index("s") * _NC + lax.axis_index("c")
        crow = wid * _NCHUNK

        pltpu.sync_copy(h_hbm.at[pl.ds(crow, _NCHUNK)], idx_h)
        pltpu.sync_copy(r_hbm.at[pl.ds(crow, _NCHUNK)], idx_r)
        pltpu.sync_copy(t_hbm.at[pl.ds(crow, _NCHUNK)], idx_t)

        copies = []
        for j in range(_NCHUNK):
            dst = pl.ds(j * _CHUNK, _CHUNK)
            copies.append(
                pltpu.async_copy(ent_hbm.at[idx_h.at[j]], rows_h.at[dst], sem))
            copies.append(
                pltpu.async_copy(rel_hbm.at[idx_r.at[j]], rows_r.at[dst], sem))
            copies.append(
                pltpu.async_copy(ent_hbm.at[idx_t.at[j]], rows_t.at[dst], sem))
        for c in copies:
            c.wait()

        lane = lax.iota(jnp.int32, 16)

        def group_body(g, carry):
            rids = g * 16 + lane
            acc = jnp.zeros((16,), jnp.float32)
            for d in range(HIDDEN_DIM):
                col = jnp.full((16,), d, jnp.int32)
                hv = plsc.load_gather(rows_h, [rids, col])
                rv = plsc.load_gather(rows_r, [rids, col])
                tv = plsc.load_gather(rows_t, [rids, col])
                acc = acc + jnp.abs(hv + rv - tv)
            out_v[pl.ds(g * 16, 16)] = GAMMA - acc
            return carry

        lax.fori_loop(0, _GROUPS, group_body, 0)

        pltpu.sync_copy(out_v, out_hbm.at[pl.ds(wid * _BPW, _BPW)])

    return kge_score


_KERNEL = _make_kernel()


def kernel(sample, entity_embedding, relation_embedding):
    h_idx = sample[:, 0].reshape(_NW * _NCHUNK, _CHUNK)
    r_idx = sample[:, 1].reshape(_NW * _NCHUNK, _CHUNK)
    t_idx = sample[:, 2].reshape(_NW * _NCHUNK, _CHUNK)
    scores = _KERNEL(h_idx, r_idx, t_idx, entity_embedding, relation_embedding)
    return scores.reshape(BATCH, 1)
